# baseline (speedup 1.0000x reference)
import functools

import jax
import jax.numpy as jnp
from jax import lax
from jax.experimental import pallas as pl
from jax.experimental.pallas import tpu as pltpu
from jax.experimental.pallas import tpu_sc as plsc

TILES = 16
CH = 1024
GRP = 128
MCAP = 4096
FLUSH = MCAP - CH - 16
L0CAP = 2048


def _rsqrt16(x):
    s = x
    for _ in range(16):
        s = 0.5 * (s + x / s)
    y = 1.0 / s
    y = y * (2.0 - s * y)
    y = y * (1.5 - 0.5 * x * y * y)
    return y


def _recip(d):
    r = 1.0 / d
    return r * (2.0 - d * r)


def _splat(ref, idxs):
    full = [jnp.full((16,), i, jnp.int32) for i in idxs]
    return plsc.load_gather(ref, full)


def _sigmoid(v):
    return _recip(1.0 + jnp.exp(-v))


def _tanh(v):
    a = jnp.abs(v)
    e = jnp.exp(-2.0 * a)
    t = (1.0 - e) * _recip(1.0 + e)
    return jnp.where(v < 0, -t, t)


def _make_sc_kernel(NPAD, NCH, D, HG):
    NSL = NPAD // TILES
    NR = 79 * GRP
    NRSL = NR // TILES
    NMW = NPAD // 16
    NMWSL = NMW // TILES
    mesh = plsc.VectorSubcoreMesh(core_axis_name="c", subcore_axis_name="s",
                                  num_cores=1)

    OW1 = 0
    OB1 = OW1 + D * HG
    OW2 = OB1 + HG
    OB2 = OW2 + HG * HG
    OBIH = OB2 + HG
    OBHH = OBIH + 4 * HG
    OWFC = OBHH + 4 * HG
    OBFC = OWFC + HG
    WTOT = ((OBFC + 16 + 127) // 128) * 128

    def body(src_r, dst_r, xt_r, wsm_r, wihT_r, out_r,
             srcst, dstst, vals1, msrc, mdst, mask_loc, maskbuf,
             zbuf, rows, idxst, idxg, dbuf, l0buf, l0val, wsmv, zvm,
             zallv, ovm,
             deg_sp, cnt0_sp, mask_sp, agg1_sp):
        sid = lax.axis_index("s")
        base = sid * NSL
        lanes = lax.iota(jnp.int32, 16)

        def wread(o):
            return wsmv[o // 128, pl.ds(o % 128, 16)]

        def ihread(o):
            return rows[o // 128, pl.ds(o % 128, 16)]

        def zb(k, _):
            zbuf[pl.ds(k * 16, 16)] = jnp.zeros((16,), jnp.float32)
            return 0
        lax.fori_loop(0, NSL // 16, zb, 0)
        pltpu.sync_copy(zbuf, deg_sp.at[pl.ds(base, NSL)])
        pltpu.sync_copy(zbuf, cnt0_sp.at[pl.ds(base, NSL)])

        def zrow(r, _):
            for c in range(D // 16):
                rows[r, pl.ds(c * 16, 16)] = jnp.zeros((16,), jnp.float32)
            return 0
        lax.fori_loop(0, GRP, zrow, 0)
        base2 = sid * NRSL
        for p in range(NRSL // GRP):
            pltpu.sync_copy(rows, agg1_sp.at[pl.ds(base2 + p * GRP, GRP)])
        tail = NRSL % GRP
        if tail:
            pltpu.sync_copy(
                rows.at[pl.ds(0, tail)],
                agg1_sp.at[pl.ds(base2 + (NRSL // GRP) * GRP, tail)])

        pltpu.sync_copy(wsm_r, wsmv)

        def ob(r, _):
            vals1[pl.ds(r * 16, 16)] = jnp.ones((16,), jnp.float32)
            return 0
        lax.fori_loop(0, CH // 16, ob, 0)

        plsc.subcore_barrier()

        def hist_chunk(c, l0n):
            pltpu.sync_copy(src_r.at[sid, c], srcst)
            pltpu.sync_copy(dst_r.at[sid, c], dstst)
            pltpu.sync_copy(vals1, deg_sp.at[dstst], add=True)

            def mk(r, l0n):
                for u in range(4):
                    rr = r * 4 + u
                    d = dstst[pl.ds(rr * 16, 16)]
                    s = srcst[pl.ds(rr * 16, 16)]
                    m = d == 0
                    off = jnp.minimum(l0n, L0CAP - 16)
                    plsc.store_compressed(
                        l0buf.at[pl.ds(off, 16)], s, mask=m)
                    n = jnp.max(plsc.all_reduce_population_count(m))
                    l0n = l0n + n
                return l0n
            return lax.fori_loop(0, CH // 64, mk, l0n)
        l0n = lax.fori_loop(0, NCH, hist_chunk, jnp.int32(0))
        l0ov = l0n > L0CAP - 16

        def cnt0_small(_):
            def pad(j, _):
                pos = j * 16 + lanes
                v = l0buf[pl.ds(j * 16, 16)]
                l0buf[pl.ds(j * 16, 16)] = jnp.where(pos < l0n, v, 0)
                l0val[pl.ds(j * 16, 16)] = jnp.where(pos < l0n, 1.0, 0.0)
                return 0
            lax.fori_loop(0, L0CAP // 16, pad, 0)
            pltpu.sync_copy(l0val, cnt0_sp.at[l0buf], add=True)
            return jnp.int32(0)

        def cnt0_rescan(_):
            def rc(c, _):
                pltpu.sync_copy(src_r.at[sid, c], srcst)
                pltpu.sync_copy(dst_r.at[sid, c], dstst)

                def mk2(r, _):
                    d = dstst[pl.ds(r * 16, 16)]
                    l0val[pl.ds(r * 16, 16)] = jnp.where(
                        d == 0, 1.0, 0.0).astype(jnp.float32)
                    return 0
                lax.fori_loop(0, CH // 16, mk2, 0)
                pltpu.sync_copy(l0val.at[pl.ds(0, CH)],
                                cnt0_sp.at[srcst], add=True)
                return 0
            lax.fori_loop(0, NCH, rc, 0)
            return jnp.int32(0)
        _ = lax.cond(l0ov, cnt0_rescan, cnt0_small, jnp.int32(0))

        plsc.subcore_barrier()

        pltpu.sync_copy(deg_sp.at[pl.ds(base, NSL)], zbuf)

        def dk(k, _):
            x = zbuf[pl.ds(k * 16, 16)] + 1.0
            zbuf[pl.ds(k * 16, 16)] = _rsqrt16(x)
            return 0
        lax.fori_loop(0, NSL // 16, dk, 0)
        pltpu.sync_copy(zbuf, deg_sp.at[pl.ds(base, NSL)])

        pltpu.sync_copy(cnt0_sp.at[pl.ds(base, NSL)], zbuf)

        def mw(k, _):
            c0 = zbuf[pl.ds(k * 16, 16)]
            bits = jnp.where(c0 > 0.0, jnp.int32(1) << lanes, 0)
            w = jnp.sum(bits)
            plsc.store_scatter(maskbuf, [jnp.full((16,), k, jnp.int32)],
                               jnp.full((16,), w, jnp.int32),
                               mask=lanes == 0)
            return 0
        lax.fori_loop(0, NMWSL, mw, 0)
        pltpu.sync_copy(maskbuf, mask_sp.at[pl.ds(sid * NMWSL, NMWSL)])
        plsc.subcore_barrier()
        pltpu.sync_copy(mask_sp, mask_loc)

        def flush4(moff):
            ngrp = (moff + GRP - 1) // GRP

            def g4(g, _):
                def cpg(j, _):
                    pos = g * GRP + j * 16 + lanes
                    v = msrc[pl.ds(g * GRP + j * 16, 16)]
                    idxg[pl.ds(j * 16, 16)] = jnp.where(pos < moff, v, 0)
                    return 0
                lax.fori_loop(0, GRP // 16, cpg, 0)
                pltpu.sync_copy(deg_sp.at[idxg], dbuf)

                def wz(j, _):
                    pos = g * GRP + j * 16 + lanes
                    dv = dbuf[pl.ds(j * 16, 16)]
                    dbuf[pl.ds(j * 16, 16)] = jnp.where(pos < moff, dv, 0.0)
                    return 0
                lax.fori_loop(0, GRP // 16, wz, 0)
                pltpu.sync_copy(xt_r.at[idxg], rows)

                def scale(rr, _):
                    dv = _splat(dbuf, [rr])
                    for cc in range(D // 16):
                        rows[rr, pl.ds(cc * 16, 16)] = (
                            rows[rr, pl.ds(cc * 16, 16)] * dv)
                    return 0
                lax.fori_loop(0, GRP, scale, 0)

                def cps(j, _):
                    pos = g * GRP + j * 16 + lanes
                    v = mdst[pl.ds(g * GRP + j * 16, 16)]
                    idxst[pl.ds(j * 16, 16)] = jnp.where(pos < moff, v, 0)
                    return 0
                lax.fori_loop(0, GRP // 16, cps, 0)
                pltpu.sync_copy(rows, agg1_sp.at[idxst], add=True)
                return 0
            lax.fori_loop(0, ngrp, g4, 0)
            return jnp.int32(0)

        def maybe_flush4(moff):
            return lax.cond(moff >= FLUSH, flush4, lambda m: m, moff)

        def chunk3(c, moff):
            pltpu.sync_copy(src_r.at[sid, c], srcst)
            pltpu.sync_copy(dst_r.at[sid, c], dstst)

            def grp3(i, off):
                for u in range(4):
                    ii = i * 4 + u
                    d = dstst[pl.ds(ii * 16, 16)]
                    s = srcst[pl.ds(ii * 16, 16)]
                    w16 = plsc.load_gather(mask_loc, [d >> 4])
                    bit = (w16 >> (d & 15)) & 1
                    mark = (bit > 0) | (d == 0)
                    plsc.store_compressed(
                        msrc.at[pl.ds(off, 16)], s, mask=mark)
                    plsc.store_compressed(
                        mdst.at[pl.ds(off, 16)], d, mask=mark)
                    n = jnp.max(plsc.all_reduce_population_count(mark))
                    off = off + n
                return off
            moff = lax.fori_loop(0, CH // 64, grp3, moff)
            return maybe_flush4(moff)
        moff = lax.fori_loop(0, NCH, chunk3, jnp.int32(0))

        pltpu.sync_copy(cnt0_sp.at[pl.ds(base, NSL)], zbuf)

        def virt(k, off):
            v = base + k * 16 + lanes
            c0 = zbuf[pl.ds(k * 16, 16)]
            mark = (c0 > 0.0) | (v == 0)
            plsc.store_compressed(msrc.at[pl.ds(off, 16)], v, mask=mark)
            plsc.store_compressed(mdst.at[pl.ds(off, 16)], v, mask=mark)
            n = jnp.max(plsc.all_reduce_population_count(mark))
            return maybe_flush4(off + n)
        moff = lax.fori_loop(0, NSL // 16, virt, moff)
        _ = lax.cond(moff > 0, flush4, lambda m: m, moff)

        plsc.subcore_barrier()

        def flush6(args):
            koff, zacc = args
            ngrp = (koff + GRP - 1) // GRP

            def g6(g, zacc):
                def cpg2(j, _):
                    pos = g * GRP + j * 16 + lanes
                    v = msrc[pl.ds(g * GRP + j * 16, 16)]
                    idxg[pl.ds(j * 16, 16)] = jnp.where(pos < koff, v, 0)
                    return 0
                lax.fori_loop(0, GRP // 16, cpg2, 0)
                pltpu.sync_copy(deg_sp.at[idxg], dbuf)

                def wz2(j, _):
                    pos = g * GRP + j * 16 + lanes
                    dv = dbuf[pl.ds(j * 16, 16)]
                    dbuf[pl.ds(j * 16, 16)] = jnp.where(pos < koff, dv, 0.0)
                    return 0
                lax.fori_loop(0, GRP // 16, wz2, 0)
                pltpu.sync_copy(agg1_sp.at[idxg], rows)

                def ent(j, zacc):
                    ds_s = _splat(dbuf, [j])

                    def mv(k, h):
                        ck = _splat(rows, [j, k])
                        return tuple(
                            h[q] + ck * wread(OW1 + k * HG + q * 16)
                            for q in range(HG // 16))
                    h = lax.fori_loop(
                        0, D, mv,
                        tuple(jnp.zeros((16,), jnp.float32)
                              for _ in range(HG // 16)))
                    out = []
                    for q in range(HG // 16):
                        h1q = jnp.maximum(
                            ds_s * h[q] + wread(OB1 + q * 16), 0.0)
                        out.append(zacc[q] + ds_s * h1q)
                    return tuple(out)
                return lax.fori_loop(
                    0, jnp.minimum(koff - g * GRP, GRP), ent, zacc)
            zacc = lax.fori_loop(0, ngrp, g6, zacc)
            return jnp.int32(0), zacc

        def maybe_flush6(args):
            return lax.cond(args[0] >= FLUSH, flush6, lambda a: a, args)

        zacc0 = tuple(jnp.zeros((16,), jnp.float32) for _ in range(HG // 16))
        plsc.store_scatter(msrc, [jnp.full((16,), 0, jnp.int32)],
                           jnp.zeros((16,), jnp.int32),
                           mask=(lanes == 0) & (sid == 0))
        k0 = jnp.where(sid == 0, jnp.int32(1), jnp.int32(0))

        def p5_small(args):
            k0, zacc = args

            def cp(j, k):
                v = l0buf[pl.ds(j * 16, 16)]
                plsc.store_compressed(
                    msrc.at[pl.ds(k, 16)], v,
                    mask=(j * 16 + lanes) < l0n)
                n = jnp.max(plsc.all_reduce_population_count(
                    (j * 16 + lanes) < l0n))
                return k + n
            koff = lax.fori_loop(0, L0CAP // 16, cp, k0)
            return flush6((koff, zacc))

        def p5_rescan(args):
            def chunk5(c, args):
                koff, zacc = args
                pltpu.sync_copy(src_r.at[sid, c], srcst)
                pltpu.sync_copy(dst_r.at[sid, c], dstst)

                def grp5(i, off):
                    d = dstst[pl.ds(i * 16, 16)]
                    s = srcst[pl.ds(i * 16, 16)]
                    m = d == 0
                    plsc.store_compressed(msrc.at[pl.ds(off, 16)], s, mask=m)
                    n = jnp.max(plsc.all_reduce_population_count(m))
                    return off + n
                koff = lax.fori_loop(0, CH // 16, grp5, koff)
                return maybe_flush6((koff, zacc))
            koff, zacc = lax.fori_loop(0, NCH, chunk5, args)
            return lax.cond(koff > 0, flush6, lambda a: a, (koff, zacc))
        _, zacc = lax.cond(l0ov, p5_rescan, p5_small, (k0, zacc0))

        for q in range(HG // 16):
            zvm[pl.ds(q * 16, 16)] = zacc[q]
        pltpu.sync_copy(zvm, cnt0_sp.at[pl.ds(sid * HG, HG)])

        plsc.subcore_barrier()

        @pl.when(sid == 0)
        def _():
            pltpu.sync_copy(cnt0_sp.at[pl.ds(0, TILES * HG)], zallv)

            def acc(t, z):
                return tuple(
                    z[q] + zallv[pl.ds(t * HG + q * 16, 16)]
                    for q in range(HG // 16))
            z = lax.fori_loop(
                0, TILES, acc,
                tuple(jnp.zeros((16,), jnp.float32)
                      for _ in range(HG // 16)))
            for q in range(HG // 16):
                zvm[pl.ds(q * 16, 16)] = z[q]

            def zi(j, _):
                idxg[pl.ds(j * 16, 16)] = jnp.zeros((16,), jnp.int32)
                return 0
            lax.fori_loop(0, GRP // 16, zi, 0)
            pltpu.sync_copy(deg_sp.at[idxg], dbuf)
            dis0 = _splat(dbuf, [0])

            def mv2(k, h):
                zk = _splat(zvm, [k])
                return tuple(
                    h[q] + zk * wread(OW2 + k * HG + q * 16)
                    for q in range(HG // 16))
            h2 = lax.fori_loop(
                0, HG, mv2,
                tuple(jnp.zeros((16,), jnp.float32)
                      for _ in range(HG // 16)))
            for q in range(HG // 16):
                zvm[pl.ds(q * 16, 16)] = (
                    dis0 * h2[q] + wread(OB2 + q * 16))

            pltpu.sync_copy(wihT_r, rows)

            def mv3(k, g):
                hk = _splat(zvm, [k])
                return tuple(
                    g[q] + hk * ihread(k * 4 * HG + q * 16)
                    for q in range(16))
            gates = lax.fori_loop(
                0, HG, mv3,
                tuple(jnp.zeros((16,), jnp.float32) for _ in range(16)))
            gates = [gates[q] + wread(OBIH + q * 16)
                     + wread(OBHH + q * 16) for q in range(16)]
            nq = HG // 16
            yacc = jnp.zeros((16,), jnp.float32)
            for q in range(nq):
                i_g = _sigmoid(gates[q])
                g_g = _tanh(gates[2 * nq + q])
                o_g = _sigmoid(gates[3 * nq + q])
                c1 = i_g * g_g
                h1l = o_g * _tanh(c1)
                yacc = yacc + h1l * wread(OWFC + q * 16)
            bfc16 = wread(OBFC)
            y = jnp.sum(yacc)
            ovm[pl.ds(0, 16)] = (jnp.where(lanes == 0, y, 0.0)
                                 + bfc16).astype(jnp.float32)
            pltpu.sync_copy(ovm, out_r)

    kern = pl.kernel(
        body,
        out_type=jax.ShapeDtypeStruct((16,), jnp.float32),
        mesh=mesh,
        compiler_params=pltpu.CompilerParams(needs_layout_passes=False),
        scratch_types=[
            pltpu.VMEM((CH,), jnp.int32),
            pltpu.VMEM((CH,), jnp.int32),
            pltpu.VMEM((CH,), jnp.float32),
            pltpu.VMEM((MCAP + GRP,), jnp.int32),
            pltpu.VMEM((MCAP + GRP,), jnp.int32),
            pltpu.VMEM((NMW,), jnp.int32),
            pltpu.VMEM((NMWSL,), jnp.int32),
            pltpu.VMEM((NSL,), jnp.float32),
            pltpu.VMEM((GRP, D), jnp.float32),
            pltpu.VMEM((GRP,), jnp.int32),
            pltpu.VMEM((GRP,), jnp.int32),
            pltpu.VMEM((GRP,), jnp.float32),
            pltpu.VMEM((L0CAP,), jnp.int32),
            pltpu.VMEM((L0CAP,), jnp.float32),
            pltpu.VMEM((WTOT // 128, 128), jnp.float32),
            pltpu.VMEM((HG,), jnp.float32),
            pltpu.VMEM((TILES * HG,), jnp.float32),
            pltpu.VMEM((16,), jnp.float32),
            pltpu.VMEM_SHARED((NPAD,), jnp.float32),
            pltpu.VMEM_SHARED((NPAD,), jnp.float32),
            pltpu.VMEM_SHARED((NMW,), jnp.int32),
            pltpu.VMEM_SHARED((NR, D), jnp.float32),
        ],
    )
    return kern


@functools.partial(jax.jit, static_argnames=())
def kernel(x, edge_index, W1, b1, W2, b2, W_ih, W_hh, b_ih, b_hh, W_fc, b_fc):
    D, N = x.shape
    E = edge_index.shape[1]
    HG = W1.shape[1]
    NPAD = ((N + 16 * 128 - 1) // (16 * 128)) * (16 * 128)
    EPK = ((E + TILES * CH - 1) // (TILES * CH)) * CH
    NCH = EPK // CH
    EPAD = EPK * TILES

    npads = EPAD - E
    sent = (N + jnp.arange(npads, dtype=jnp.int32) % (NPAD - N)).astype(
        edge_index.dtype)
    src = jnp.concatenate([edge_index[0], sent]).reshape(TILES, NCH, CH)
    dst = jnp.concatenate([edge_index[1], sent]).reshape(TILES, NCH, CH)
    xt = jnp.zeros((NPAD, D), x.dtype).at[:N].set(x.T)
    wsm = jnp.concatenate([
        W1.reshape(-1), b1, W2.reshape(-1), b2,
        b_ih, b_hh, W_fc[0], jnp.pad(b_fc, (0, 15))]).astype(jnp.float32)
    wtot = ((wsm.shape[0] + 127) // 128) * 128
    wsm = jnp.pad(wsm, (0, wtot - wsm.shape[0])).reshape(-1, 128)
    wihT = W_ih.T.astype(jnp.float32).reshape(GRP, -1)

    out = _make_sc_kernel(NPAD, NCH, D, HG)(src, dst, xt, wsm, wihT)
    return out[0].reshape(1, 1, 1)


# speedup vs baseline: 1.0075x; 1.0075x over previous
import functools

import jax
import jax.numpy as jnp
from jax import lax
from jax.experimental import pallas as pl
from jax.experimental.pallas import tpu as pltpu
from jax.experimental.pallas import tpu_sc as plsc

TILES = 16
CH = 1024
GRP = 128
MCAP = 4096
FLUSH = MCAP - CH - 16
L0CAP = 2048


def _rsqrt16(x):
    s = x
    for _ in range(16):
        s = 0.5 * (s + x / s)
    y = 1.0 / s
    y = y * (2.0 - s * y)
    y = y * (1.5 - 0.5 * x * y * y)
    return y


def _recip(d):
    r = 1.0 / d
    return r * (2.0 - d * r)


def _splat(ref, idxs):
    full = [jnp.full((16,), i, jnp.int32) for i in idxs]
    return plsc.load_gather(ref, full)


def _sigmoid(v):
    return _recip(1.0 + jnp.exp(-v))


def _tanh(v):
    a = jnp.abs(v)
    e = jnp.exp(-2.0 * a)
    t = (1.0 - e) * _recip(1.0 + e)
    return jnp.where(v < 0, -t, t)


def _make_sc_kernel(NPAD, NCH, D, HG):
    NSL = NPAD // TILES
    NR = 79 * GRP
    NRSL = NR // TILES
    NMW = NPAD // 16
    NMWSL = NMW // TILES
    mesh = plsc.VectorSubcoreMesh(core_axis_name="c", subcore_axis_name="s",
                                  num_cores=1)

    OW1 = 0
    OB1 = OW1 + D * HG
    OW2 = OB1 + HG
    OB2 = OW2 + HG * HG
    OBIH = OB2 + HG
    OBHH = OBIH + 4 * HG
    OWFC = OBHH + 4 * HG
    OBFC = OWFC + HG
    WTOT = ((OBFC + 16 + 127) // 128) * 128

    def body(src_r, dst_r, xt_r, wsm_r, wihT_r, out_r,
             srcst, dstst, vals1, msrc, mdst, mask_loc, maskbuf,
             zbuf, rows, idxst, idxg, dbuf, l0buf, l0val, wsmv, zvm,
             zallv, ovm,
             deg_sp, cnt0_sp, mask_sp, agg1_sp):
        sid = lax.axis_index("s")
        base = sid * NSL
        lanes = lax.iota(jnp.int32, 16)

        def wread(o):
            return wsmv[o // 128, pl.ds(o % 128, 16)]

        def ihread(o):
            return rows[o // 128, pl.ds(o % 128, 16)]

        def zb(k, _):
            zbuf[pl.ds(k * 16, 16)] = jnp.zeros((16,), jnp.float32)
            return 0
        lax.fori_loop(0, NSL // 16, zb, 0)
        pltpu.sync_copy(zbuf, deg_sp.at[pl.ds(base, NSL)])
        pltpu.sync_copy(zbuf, cnt0_sp.at[pl.ds(base, NSL)])

        def zrow(r, _):
            for c in range(D // 16):
                rows[r, pl.ds(c * 16, 16)] = jnp.zeros((16,), jnp.float32)
            return 0
        lax.fori_loop(0, GRP, zrow, 0)
        base2 = sid * NRSL
        for p in range(NRSL // GRP):
            pltpu.sync_copy(rows, agg1_sp.at[pl.ds(base2 + p * GRP, GRP)])
        tail = NRSL % GRP
        if tail:
            pltpu.sync_copy(
                rows.at[pl.ds(0, tail)],
                agg1_sp.at[pl.ds(base2 + (NRSL // GRP) * GRP, tail)])

        pltpu.sync_copy(wsm_r, wsmv)

        def ob(r, _):
            vals1[pl.ds(r * 16, 16)] = jnp.ones((16,), jnp.float32)
            return 0
        lax.fori_loop(0, CH // 16, ob, 0)

        plsc.subcore_barrier()

        def hist_chunk(c, l0n):
            pltpu.sync_copy(src_r.at[sid, c], srcst)
            pltpu.sync_copy(dst_r.at[sid, c], dstst)
            pltpu.sync_copy(vals1, deg_sp.at[dstst], add=True)

            def mk(r, l0n):
                d = dstst[pl.ds(r * 16, 16)]
                s = srcst[pl.ds(r * 16, 16)]
                m = d == 0
                off = jnp.minimum(l0n, L0CAP - 16)
                plsc.store_compressed(l0buf.at[pl.ds(off, 16)], s, mask=m)
                n = jnp.max(plsc.all_reduce_population_count(m))
                return l0n + n
            return lax.fori_loop(0, CH // 16, mk, l0n)
        l0n = lax.fori_loop(0, NCH, hist_chunk, jnp.int32(0))
        l0ov = l0n > L0CAP - 16

        def cnt0_small(_):
            def pad(j, _):
                pos = j * 16 + lanes
                v = l0buf[pl.ds(j * 16, 16)]
                l0buf[pl.ds(j * 16, 16)] = jnp.where(pos < l0n, v, 0)
                l0val[pl.ds(j * 16, 16)] = jnp.where(pos < l0n, 1.0, 0.0)
                return 0
            lax.fori_loop(0, L0CAP // 16, pad, 0)
            pltpu.sync_copy(l0val, cnt0_sp.at[l0buf], add=True)
            return jnp.int32(0)

        def cnt0_rescan(_):
            def rc(c, _):
                pltpu.sync_copy(src_r.at[sid, c], srcst)
                pltpu.sync_copy(dst_r.at[sid, c], dstst)

                def mk2(r, _):
                    d = dstst[pl.ds(r * 16, 16)]
                    l0val[pl.ds(r * 16, 16)] = jnp.where(
                        d == 0, 1.0, 0.0).astype(jnp.float32)
                    return 0
                lax.fori_loop(0, CH // 16, mk2, 0)
                pltpu.sync_copy(l0val.at[pl.ds(0, CH)],
                                cnt0_sp.at[srcst], add=True)
                return 0
            lax.fori_loop(0, NCH, rc, 0)
            return jnp.int32(0)
        _ = lax.cond(l0ov, cnt0_rescan, cnt0_small, jnp.int32(0))

        plsc.subcore_barrier()

        pltpu.sync_copy(deg_sp.at[pl.ds(base, NSL)], zbuf)

        def dk(k, _):
            x = zbuf[pl.ds(k * 16, 16)] + 1.0
            zbuf[pl.ds(k * 16, 16)] = _rsqrt16(x)
            return 0
        lax.fori_loop(0, NSL // 16, dk, 0)
        pltpu.sync_copy(zbuf, deg_sp.at[pl.ds(base, NSL)])

        pltpu.sync_copy(cnt0_sp.at[pl.ds(base, NSL)], zbuf)

        def mw(k, _):
            c0 = zbuf[pl.ds(k * 16, 16)]
            bits = jnp.where(c0 > 0.0, jnp.int32(1) << lanes, 0)
            w = jnp.sum(bits)
            plsc.store_scatter(maskbuf, [jnp.full((16,), k, jnp.int32)],
                               jnp.full((16,), w, jnp.int32),
                               mask=lanes == 0)
            return 0
        lax.fori_loop(0, NMWSL, mw, 0)
        pltpu.sync_copy(maskbuf, mask_sp.at[pl.ds(sid * NMWSL, NMWSL)])
        plsc.subcore_barrier()
        pltpu.sync_copy(mask_sp, mask_loc)

        def flush4(moff):
            ngrp = (moff + GRP - 1) // GRP

            def g4(g, _):
                def cpg(j, _):
                    pos = g * GRP + j * 16 + lanes
                    v = msrc[pl.ds(g * GRP + j * 16, 16)]
                    idxg[pl.ds(j * 16, 16)] = jnp.where(pos < moff, v, 0)
                    return 0
                lax.fori_loop(0, GRP // 16, cpg, 0)
                pltpu.sync_copy(deg_sp.at[idxg], dbuf)

                def wz(j, _):
                    pos = g * GRP + j * 16 + lanes
                    dv = dbuf[pl.ds(j * 16, 16)]
                    dbuf[pl.ds(j * 16, 16)] = jnp.where(pos < moff, dv, 0.0)
                    return 0
                lax.fori_loop(0, GRP // 16, wz, 0)
                pltpu.sync_copy(xt_r.at[idxg], rows)

                def scale(rr, _):
                    dv = _splat(dbuf, [rr])
                    for cc in range(D // 16):
                        rows[rr, pl.ds(cc * 16, 16)] = (
                            rows[rr, pl.ds(cc * 16, 16)] * dv)
                    return 0
                lax.fori_loop(0, GRP, scale, 0)

                def cps(j, _):
                    pos = g * GRP + j * 16 + lanes
                    v = mdst[pl.ds(g * GRP + j * 16, 16)]
                    idxst[pl.ds(j * 16, 16)] = jnp.where(pos < moff, v, 0)
                    return 0
                lax.fori_loop(0, GRP // 16, cps, 0)
                pltpu.sync_copy(rows, agg1_sp.at[idxst], add=True)
                return 0
            lax.fori_loop(0, ngrp, g4, 0)
            return jnp.int32(0)

        def maybe_flush4(moff):
            return lax.cond(moff >= FLUSH, flush4, lambda m: m, moff)

        def chunk3(c, moff):
            pltpu.sync_copy(src_r.at[sid, c], srcst)
            pltpu.sync_copy(dst_r.at[sid, c], dstst)

            def grp3(i, off):
                d = dstst[pl.ds(i * 16, 16)]
                s = srcst[pl.ds(i * 16, 16)]
                w16 = plsc.load_gather(mask_loc, [d >> 4])
                bit = (w16 >> (d & 15)) & 1
                mark = (bit > 0) | (d == 0)
                plsc.store_compressed(msrc.at[pl.ds(off, 16)], s, mask=mark)
                plsc.store_compressed(mdst.at[pl.ds(off, 16)], d, mask=mark)
                n = jnp.max(plsc.all_reduce_population_count(mark))
                return off + n
            moff = lax.fori_loop(0, CH // 16, grp3, moff)
            return maybe_flush4(moff)
        moff = lax.fori_loop(0, NCH, chunk3, jnp.int32(0))

        pltpu.sync_copy(cnt0_sp.at[pl.ds(base, NSL)], zbuf)

        def virt(k, off):
            v = base + k * 16 + lanes
            c0 = zbuf[pl.ds(k * 16, 16)]
            mark = (c0 > 0.0) | (v == 0)
            plsc.store_compressed(msrc.at[pl.ds(off, 16)], v, mask=mark)
            plsc.store_compressed(mdst.at[pl.ds(off, 16)], v, mask=mark)
            n = jnp.max(plsc.all_reduce_population_count(mark))
            return maybe_flush4(off + n)
        moff = lax.fori_loop(0, NSL // 16, virt, moff)
        _ = lax.cond(moff > 0, flush4, lambda m: m, moff)

        plsc.subcore_barrier()

        def flush6(args):
            koff, zacc = args
            ngrp = (koff + GRP - 1) // GRP

            def g6(g, zacc):
                def cpg2(j, _):
                    pos = g * GRP + j * 16 + lanes
                    v = msrc[pl.ds(g * GRP + j * 16, 16)]
                    idxg[pl.ds(j * 16, 16)] = jnp.where(pos < koff, v, 0)
                    return 0
                lax.fori_loop(0, GRP // 16, cpg2, 0)
                pltpu.sync_copy(deg_sp.at[idxg], dbuf)

                def wz2(j, _):
                    pos = g * GRP + j * 16 + lanes
                    dv = dbuf[pl.ds(j * 16, 16)]
                    dbuf[pl.ds(j * 16, 16)] = jnp.where(pos < koff, dv, 0.0)
                    return 0
                lax.fori_loop(0, GRP // 16, wz2, 0)
                pltpu.sync_copy(agg1_sp.at[idxg], rows)

                def ent(j, zacc):
                    ds_s = _splat(dbuf, [j])

                    def mv(k, h):
                        ck = _splat(rows, [j, k])
                        return tuple(
                            h[q] + ck * wread(OW1 + k * HG + q * 16)
                            for q in range(HG // 16))
                    h = lax.fori_loop(
                        0, D, mv,
                        tuple(jnp.zeros((16,), jnp.float32)
                              for _ in range(HG // 16)))
                    out = []
                    for q in range(HG // 16):
                        h1q = jnp.maximum(
                            ds_s * h[q] + wread(OB1 + q * 16), 0.0)
                        out.append(zacc[q] + ds_s * h1q)
                    return tuple(out)
                return lax.fori_loop(
                    0, jnp.minimum(koff - g * GRP, GRP), ent, zacc)
            zacc = lax.fori_loop(0, ngrp, g6, zacc)
            return jnp.int32(0), zacc

        def maybe_flush6(args):
            return lax.cond(args[0] >= FLUSH, flush6, lambda a: a, args)

        zacc0 = tuple(jnp.zeros((16,), jnp.float32) for _ in range(HG // 16))
        plsc.store_scatter(msrc, [jnp.full((16,), 0, jnp.int32)],
                           jnp.zeros((16,), jnp.int32),
                           mask=(lanes == 0) & (sid == 0))
        k0 = jnp.where(sid == 0, jnp.int32(1), jnp.int32(0))

        def p5_small(args):
            k0, zacc = args

            def cp(j, k):
                v = l0buf[pl.ds(j * 16, 16)]
                plsc.store_compressed(
                    msrc.at[pl.ds(k, 16)], v,
                    mask=(j * 16 + lanes) < l0n)
                n = jnp.max(plsc.all_reduce_population_count(
                    (j * 16 + lanes) < l0n))
                return k + n
            koff = lax.fori_loop(0, L0CAP // 16, cp, k0)
            return flush6((koff, zacc))

        def p5_rescan(args):
            def chunk5(c, args):
                koff, zacc = args
                pltpu.sync_copy(src_r.at[sid, c], srcst)
                pltpu.sync_copy(dst_r.at[sid, c], dstst)

                def grp5(i, off):
                    d = dstst[pl.ds(i * 16, 16)]
                    s = srcst[pl.ds(i * 16, 16)]
                    m = d == 0
                    plsc.store_compressed(msrc.at[pl.ds(off, 16)], s, mask=m)
                    n = jnp.max(plsc.all_reduce_population_count(m))
                    return off + n
                koff = lax.fori_loop(0, CH // 16, grp5, koff)
                return maybe_flush6((koff, zacc))
            koff, zacc = lax.fori_loop(0, NCH, chunk5, args)
            return lax.cond(koff > 0, flush6, lambda a: a, (koff, zacc))
        _, zacc = lax.cond(l0ov, p5_rescan, p5_small, (k0, zacc0))

        for q in range(HG // 16):
            zvm[pl.ds(q * 16, 16)] = zacc[q]
        pltpu.sync_copy(zvm, cnt0_sp.at[pl.ds(sid * HG, HG)])

        plsc.subcore_barrier()

        @pl.when(sid == 0)
        def _():
            pltpu.sync_copy(cnt0_sp.at[pl.ds(0, TILES * HG)], zallv)

            def acc(t, z):
                return tuple(
                    z[q] + zallv[pl.ds(t * HG + q * 16, 16)]
                    for q in range(HG // 16))
            z = lax.fori_loop(
                0, TILES, acc,
                tuple(jnp.zeros((16,), jnp.float32)
                      for _ in range(HG // 16)))
            for q in range(HG // 16):
                zvm[pl.ds(q * 16, 16)] = z[q]

            def zi(j, _):
                idxg[pl.ds(j * 16, 16)] = jnp.zeros((16,), jnp.int32)
                return 0
            lax.fori_loop(0, GRP // 16, zi, 0)
            pltpu.sync_copy(deg_sp.at[idxg], dbuf)
            dis0 = _splat(dbuf, [0])

            def mv2(k, h):
                zk = _splat(zvm, [k])
                return tuple(
                    h[q] + zk * wread(OW2 + k * HG + q * 16)
                    for q in range(HG // 16))
            h2 = lax.fori_loop(
                0, HG, mv2,
                tuple(jnp.zeros((16,), jnp.float32)
                      for _ in range(HG // 16)))
            for q in range(HG // 16):
                zvm[pl.ds(q * 16, 16)] = (
                    dis0 * h2[q] + wread(OB2 + q * 16))

            pltpu.sync_copy(wihT_r, rows)

            def mv3(k, g):
                hk = _splat(zvm, [k])
                return tuple(
                    g[q] + hk * ihread(k * 4 * HG + q * 16)
                    for q in range(16))
            gates = lax.fori_loop(
                0, HG, mv3,
                tuple(jnp.zeros((16,), jnp.float32) for _ in range(16)))
            gates = [gates[q] + wread(OBIH + q * 16)
                     + wread(OBHH + q * 16) for q in range(16)]
            nq = HG // 16
            yacc = jnp.zeros((16,), jnp.float32)
            for q in range(nq):
                i_g = _sigmoid(gates[q])
                g_g = _tanh(gates[2 * nq + q])
                o_g = _sigmoid(gates[3 * nq + q])
                c1 = i_g * g_g
                h1l = o_g * _tanh(c1)
                yacc = yacc + h1l * wread(OWFC + q * 16)
            bfc16 = wread(OBFC)
            y = jnp.sum(yacc)
            ovm[pl.ds(0, 16)] = (jnp.where(lanes == 0, y, 0.0)
                                 + bfc16).astype(jnp.float32)
            pltpu.sync_copy(ovm, out_r)

    kern = pl.kernel(
        body,
        out_type=jax.ShapeDtypeStruct((16,), jnp.float32),
        mesh=mesh,
        compiler_params=pltpu.CompilerParams(needs_layout_passes=False),
        scratch_types=[
            pltpu.VMEM((CH,), jnp.int32),
            pltpu.VMEM((CH,), jnp.int32),
            pltpu.VMEM((CH,), jnp.float32),
            pltpu.VMEM((MCAP + GRP,), jnp.int32),
            pltpu.VMEM((MCAP + GRP,), jnp.int32),
            pltpu.VMEM((NMW,), jnp.int32),
            pltpu.VMEM((NMWSL,), jnp.int32),
            pltpu.VMEM((NSL,), jnp.float32),
            pltpu.VMEM((GRP, D), jnp.float32),
            pltpu.VMEM((GRP,), jnp.int32),
            pltpu.VMEM((GRP,), jnp.int32),
            pltpu.VMEM((GRP,), jnp.float32),
            pltpu.VMEM((L0CAP,), jnp.int32),
            pltpu.VMEM((L0CAP,), jnp.float32),
            pltpu.VMEM((WTOT // 128, 128), jnp.float32),
            pltpu.VMEM((HG,), jnp.float32),
            pltpu.VMEM((TILES * HG,), jnp.float32),
            pltpu.VMEM((16,), jnp.float32),
            pltpu.VMEM_SHARED((NPAD,), jnp.float32),
            pltpu.VMEM_SHARED((NPAD,), jnp.float32),
            pltpu.VMEM_SHARED((NMW,), jnp.int32),
            pltpu.VMEM_SHARED((NR, D), jnp.float32),
        ],
    )
    return kern


@functools.partial(jax.jit, static_argnames=())
def kernel(x, edge_index, W1, b1, W2, b2, W_ih, W_hh, b_ih, b_hh, W_fc, b_fc):
    D, N = x.shape
    E = edge_index.shape[1]
    HG = W1.shape[1]
    NPAD = ((N + 16 * 128 - 1) // (16 * 128)) * (16 * 128)
    EPK = ((E + TILES * CH - 1) // (TILES * CH)) * CH
    NCH = EPK // CH
    EPAD = EPK * TILES

    npads = EPAD - E
    sent = (N + jnp.arange(npads, dtype=jnp.int32) % (NPAD - N)).astype(
        edge_index.dtype)
    src = jnp.concatenate([edge_index[0], sent]).reshape(TILES, NCH, CH)
    dst = jnp.concatenate([edge_index[1], sent]).reshape(TILES, NCH, CH)
    xt = jnp.zeros((NPAD, D), x.dtype).at[:N].set(x.T)
    wsm = jnp.concatenate([
        W1.reshape(-1), b1, W2.reshape(-1), b2,
        b_ih, b_hh, W_fc[0], jnp.pad(b_fc, (0, 15))]).astype(jnp.float32)
    wtot = ((wsm.shape[0] + 127) // 128) * 128
    wsm = jnp.pad(wsm, (0, wtot - wsm.shape[0])).reshape(-1, 128)
    wihT = W_ih.T.astype(jnp.float32).reshape(GRP, -1)

    out = _make_sc_kernel(NPAD, NCH, D, HG)(src, dst, xt, wsm, wihT)
    return out[0].reshape(1, 1, 1)
